# baseline (device time: 43854 ns/iter reference)
import jax
import jax.numpy as jnp
from jax import lax
from jax.experimental import pallas as pl
from jax.experimental.pallas import tpu as pltpu

N_DEV = 8
CHUNK = 2
N_CHUNKS = N_DEV // CHUNK


def kernel(x, w_mat):
    m, k = x.shape
    n = w_mat.shape[1]
    blk = n // N_DEV
    cw = CHUNK * blk

    def body(x_ref, w_ref, out_ref, send_ref, recv_ref,
             amax_send_ref, amax_all_ref, copy_sem,
             data_send_sems, data_recv_sems, amax_send_sems, amax_recv_sems):
        my = lax.axis_index("i")
        my_grp = my // CHUNK

        barrier_sem = pltpu.get_barrier_semaphore()
        for t in range(1, N_DEV):
            pl.semaphore_signal(
                barrier_sem, inc=1,
                device_id=((my + t) % N_DEV,),
                device_id_type=pl.DeviceIdType.MESH,
            )

        local_amax = jnp.float32(0.0)
        for cp_ in range(N_CHUNKS):
            c = (my_grp + cp_) % N_CHUNKS
            yc = jnp.dot(
                x_ref[...], w_ref[:, pl.ds(c * cw, cw)],
                preferred_element_type=jnp.float32,
            )
            local_amax = jnp.maximum(local_amax, jnp.max(jnp.abs(yc)))
            ybf = yc.astype(jnp.bfloat16)
            for u in range(CHUNK):
                j = c * CHUNK + u
                t = (j - my) % N_DEV
                send_ref[pl.ds(j, 1), :, :] = ybf[:, u * blk:(u + 1) * blk][
                    None, :, :
                ]
                if cp_ == 0 and u == 0:
                    pl.semaphore_wait(barrier_sem, N_DEV - 1)

                @pl.when(j != my)
                def _(j=j, t=t):
                    rdma = pltpu.make_async_remote_copy(
                        src_ref=send_ref.at[j],
                        dst_ref=recv_ref.at[t],
                        send_sem=data_send_sems.at[j],
                        recv_sem=data_recv_sems.at[t],
                        device_id=(j,),
                        device_id_type=pl.DeviceIdType.MESH,
                    )
                    rdma.start()

                @pl.when(j == my)
                def _(j=j):
                    cp2 = pltpu.make_async_copy(
                        send_ref.at[j], recv_ref.at[0], copy_sem
                    )
                    cp2.start()
                    cp2.wait()

        amax_send_ref[...] = jnp.full((1, 128), local_amax, jnp.float32)
        amax_all_ref[pl.ds(0, 1), :] = amax_send_ref[...]
        amax_rdmas = []
        for t in range(1, N_DEV):
            rdma = pltpu.make_async_remote_copy(
                src_ref=amax_send_ref,
                dst_ref=amax_all_ref.at[pl.ds(t, 1), :],
                send_sem=amax_send_sems.at[t],
                recv_sem=amax_recv_sems.at[t],
                device_id=((my + t) % N_DEV,),
                device_id_type=pl.DeviceIdType.MESH,
            )
            rdma.start()
            amax_rdmas.append(rdma)

        for t in range(N_DEV):
            if t > 0:
                pltpu.make_async_remote_copy(
                    src_ref=send_ref.at[0],
                    dst_ref=recv_ref.at[t],
                    send_sem=data_send_sems.at[0],
                    recv_sem=data_recv_sems.at[t],
                    device_id=(0,),
                    device_id_type=pl.DeviceIdType.MESH,
                ).wait_recv()
            src = (my + N_DEV - t) % N_DEV
            out_ref[pl.ds(src * m, m), :] = recv_ref[t, :, :].astype(
                jnp.float32
            )

        for rdma in amax_rdmas:
            rdma.wait_recv()
        gmax = jnp.max(amax_all_ref[...])
        scale = gmax / 127.0
        inv_scale = 127.0 / gmax
        out_ref[...] = (
            jnp.clip(jnp.round(out_ref[...] * inv_scale), -127.0, 127.0)
            * scale
        )

        for j in range(N_DEV):
            @pl.when(j != my)
            def _(j=j):
                pltpu.make_async_remote_copy(
                    src_ref=send_ref.at[j],
                    dst_ref=recv_ref.at[0],
                    send_sem=data_send_sems.at[j],
                    recv_sem=data_recv_sems.at[0],
                    device_id=(0,),
                    device_id_type=pl.DeviceIdType.MESH,
                ).wait_send()
        for rdma in amax_rdmas:
            rdma.wait_send()

    out_shape = jax.ShapeDtypeStruct((N_DEV * m, blk), jnp.float32)
    return pl.pallas_call(
        body,
        out_shape=out_shape,
        in_specs=[
            pl.BlockSpec(memory_space=pltpu.VMEM),
            pl.BlockSpec(memory_space=pltpu.VMEM),
        ],
        out_specs=pl.BlockSpec(memory_space=pltpu.VMEM),
        scratch_shapes=[
            pltpu.VMEM((N_DEV, m, blk), jnp.bfloat16),
            pltpu.VMEM((N_DEV, m, blk), jnp.bfloat16),
            pltpu.VMEM((1, 128), jnp.float32),
            pltpu.VMEM((N_DEV, 128), jnp.float32),
            pltpu.SemaphoreType.DMA,
            pltpu.SemaphoreType.DMA((N_DEV,)),
            pltpu.SemaphoreType.DMA((N_DEV,)),
            pltpu.SemaphoreType.DMA((N_DEV,)),
            pltpu.SemaphoreType.DMA((N_DEV,)),
        ],
        compiler_params=pltpu.CompilerParams(
            vmem_limit_bytes=60 * 1024 * 1024,
            collective_id=0,
        ),
    )(x, w_mat)


# device time: 43612 ns/iter; 1.0055x vs baseline; 1.0055x over previous
import jax
import jax.numpy as jnp
from jax import lax
from jax.experimental import pallas as pl
from jax.experimental.pallas import tpu as pltpu

N_DEV = 8
CHUNK = 2
N_CHUNKS = N_DEV // CHUNK


def kernel(x, w_mat):
    m, k = x.shape
    n = w_mat.shape[1]
    blk = n // N_DEV
    cw = CHUNK * blk

    def body(x_ref, w_ref, out_ref, send_ref, recv_ref,
             amax_send_ref, amax_all_ref, copy_sem,
             data_send_sems, data_recv_sems, amax_send_sems, amax_recv_sems):
        my = lax.axis_index("i")
        my_grp = my // CHUNK

        barrier_sem = pltpu.get_barrier_semaphore()
        for t in range(1, N_DEV):
            pl.semaphore_signal(
                barrier_sem, inc=1,
                device_id=((my + t) % N_DEV,),
                device_id_type=pl.DeviceIdType.MESH,
            )

        local_amax = jnp.float32(0.0)
        for cp_ in range(N_CHUNKS):
            c = (my_grp + cp_) % N_CHUNKS
            yc = jnp.dot(
                x_ref[...], w_ref[:, pl.ds(c * cw, cw)],
                preferred_element_type=jnp.float32,
            )
            local_amax = jnp.maximum(local_amax, jnp.max(jnp.abs(yc)))
            ybf = yc.astype(jnp.bfloat16)
            for u in range(CHUNK):
                j = c * CHUNK + u
                t = (j - my) % N_DEV
                send_ref[pl.ds(j, 1), :, :] = ybf[:, u * blk:(u + 1) * blk][
                    None, :, :
                ]
                if cp_ == 0 and u == 0:
                    pl.semaphore_wait(barrier_sem, N_DEV - 1)

                @pl.when(j != my)
                def _(j=j, t=t):
                    rdma = pltpu.make_async_remote_copy(
                        src_ref=send_ref.at[j],
                        dst_ref=recv_ref.at[t],
                        send_sem=data_send_sems.at[j],
                        recv_sem=data_recv_sems.at[t],
                        device_id=(j,),
                        device_id_type=pl.DeviceIdType.MESH,
                    )
                    rdma.start()

                @pl.when(j == my)
                def _(j=j):
                    cp2 = pltpu.make_async_copy(
                        send_ref.at[j], recv_ref.at[0], copy_sem
                    )
                    cp2.start()
                    cp2.wait()

        amax_send_ref[...] = jnp.full((1, 128), local_amax, jnp.float32)
        amax_all_ref[pl.ds(0, 1), :] = amax_send_ref[...]
        amax_rdmas = []
        for t in range(1, N_DEV):
            rdma = pltpu.make_async_remote_copy(
                src_ref=amax_send_ref,
                dst_ref=amax_all_ref.at[pl.ds(t, 1), :],
                send_sem=amax_send_sems.at[t],
                recv_sem=amax_recv_sems.at[t],
                device_id=((my + t) % N_DEV,),
                device_id_type=pl.DeviceIdType.MESH,
            )
            rdma.start()
            amax_rdmas.append(rdma)

        for rdma in amax_rdmas:
            rdma.wait_recv()
        gmax = jnp.max(amax_all_ref[...])
        scale = gmax / 127.0
        inv_scale = 127.0 / gmax
        for t in range(N_DEV):
            if t > 0:
                pltpu.make_async_remote_copy(
                    src_ref=send_ref.at[0],
                    dst_ref=recv_ref.at[t],
                    send_sem=data_send_sems.at[0],
                    recv_sem=data_recv_sems.at[t],
                    device_id=(0,),
                    device_id_type=pl.DeviceIdType.MESH,
                ).wait_recv()
            src = (my + N_DEV - t) % N_DEV
            q = jnp.clip(
                jnp.round(recv_ref[t, :, :].astype(jnp.float32) * inv_scale),
                -127.0, 127.0,
            )
            out_ref[pl.ds(src * m, m), :] = q * scale

        for j in range(N_DEV):
            @pl.when(j != my)
            def _(j=j):
                pltpu.make_async_remote_copy(
                    src_ref=send_ref.at[j],
                    dst_ref=recv_ref.at[0],
                    send_sem=data_send_sems.at[j],
                    recv_sem=data_recv_sems.at[0],
                    device_id=(0,),
                    device_id_type=pl.DeviceIdType.MESH,
                ).wait_send()
        for rdma in amax_rdmas:
            rdma.wait_send()

    out_shape = jax.ShapeDtypeStruct((N_DEV * m, blk), jnp.float32)
    return pl.pallas_call(
        body,
        out_shape=out_shape,
        in_specs=[
            pl.BlockSpec(memory_space=pltpu.VMEM),
            pl.BlockSpec(memory_space=pltpu.VMEM),
        ],
        out_specs=pl.BlockSpec(memory_space=pltpu.VMEM),
        scratch_shapes=[
            pltpu.VMEM((N_DEV, m, blk), jnp.bfloat16),
            pltpu.VMEM((N_DEV, m, blk), jnp.bfloat16),
            pltpu.VMEM((1, 128), jnp.float32),
            pltpu.VMEM((N_DEV, 128), jnp.float32),
            pltpu.SemaphoreType.DMA,
            pltpu.SemaphoreType.DMA((N_DEV,)),
            pltpu.SemaphoreType.DMA((N_DEV,)),
            pltpu.SemaphoreType.DMA((N_DEV,)),
            pltpu.SemaphoreType.DMA((N_DEV,)),
        ],
        compiler_params=pltpu.CompilerParams(
            vmem_limit_bytes=60 * 1024 * 1024,
            collective_id=0,
        ),
    )(x, w_mat)


# device time: 43493 ns/iter; 1.0083x vs baseline; 1.0027x over previous
import jax
import jax.numpy as jnp
from jax import lax
from jax.experimental import pallas as pl
from jax.experimental.pallas import tpu as pltpu

N_DEV = 8
CHUNK = 2
N_CHUNKS = N_DEV // CHUNK


def kernel(x, w_mat):
    m, k = x.shape
    n = w_mat.shape[1]
    blk = n // N_DEV
    cw = CHUNK * blk

    def body(x_ref, w_ref, out_ref, send_ref, recv_ref,
             amax_send_ref, amax_all_ref, copy_sem,
             data_send_sems, data_recv_sems, amax_send_sems, amax_recv_sems):
        my = lax.axis_index("i")

        barrier_sem = pltpu.get_barrier_semaphore()
        for t in range(1, N_DEV):
            pl.semaphore_signal(
                barrier_sem, inc=1,
                device_id=((my + t) % N_DEV,),
                device_id_type=pl.DeviceIdType.MESH,
            )

        local_amax = jnp.float32(0.0)
        for cp_ in range(N_CHUNKS):
            c = cp_
            yc = jnp.dot(
                x_ref[...], w_ref[:, c * cw:(c + 1) * cw],
                preferred_element_type=jnp.float32,
            )
            local_amax = jnp.maximum(local_amax, jnp.max(jnp.abs(yc)))
            ybf = yc.astype(jnp.bfloat16)
            for u in range(CHUNK):
                j = c * CHUNK + u
                t = (j - my) % N_DEV
                send_ref[j, :, :] = ybf[:, u * blk:(u + 1) * blk]
                if cp_ == 0 and u == 0:
                    pl.semaphore_wait(barrier_sem, N_DEV - 1)

                @pl.when(j != my)
                def _(j=j, t=t):
                    rdma = pltpu.make_async_remote_copy(
                        src_ref=send_ref.at[j],
                        dst_ref=recv_ref.at[t],
                        send_sem=data_send_sems.at[j],
                        recv_sem=data_recv_sems.at[t],
                        device_id=(j,),
                        device_id_type=pl.DeviceIdType.MESH,
                    )
                    rdma.start()

                @pl.when(j == my)
                def _(j=j):
                    cp2 = pltpu.make_async_copy(
                        send_ref.at[j], recv_ref.at[0], copy_sem
                    )
                    cp2.start()
                    cp2.wait()

        amax_send_ref[...] = jnp.full((1, 128), local_amax, jnp.float32)
        amax_all_ref[pl.ds(0, 1), :] = amax_send_ref[...]
        amax_rdmas = []
        for t in range(1, N_DEV):
            rdma = pltpu.make_async_remote_copy(
                src_ref=amax_send_ref,
                dst_ref=amax_all_ref.at[pl.ds(t, 1), :],
                send_sem=amax_send_sems.at[t],
                recv_sem=amax_recv_sems.at[t],
                device_id=((my + t) % N_DEV,),
                device_id_type=pl.DeviceIdType.MESH,
            )
            rdma.start()
            amax_rdmas.append(rdma)

        for rdma in amax_rdmas:
            rdma.wait_recv()
        gmax = jnp.max(amax_all_ref[...])
        scale = gmax / 127.0
        inv_scale = 127.0 / gmax
        for t in range(N_DEV):
            if t > 0:
                pltpu.make_async_remote_copy(
                    src_ref=send_ref.at[0],
                    dst_ref=recv_ref.at[t],
                    send_sem=data_send_sems.at[0],
                    recv_sem=data_recv_sems.at[t],
                    device_id=(0,),
                    device_id_type=pl.DeviceIdType.MESH,
                ).wait_recv()
            src = (my + N_DEV - t) % N_DEV
            q = jnp.clip(
                jnp.round(recv_ref[t, :, :].astype(jnp.float32) * inv_scale),
                -127.0, 127.0,
            )
            out_ref[pl.ds(src * m, m), :] = q * scale

        for j in range(N_DEV):
            @pl.when(j != my)
            def _(j=j):
                pltpu.make_async_remote_copy(
                    src_ref=send_ref.at[j],
                    dst_ref=recv_ref.at[0],
                    send_sem=data_send_sems.at[j],
                    recv_sem=data_recv_sems.at[0],
                    device_id=(0,),
                    device_id_type=pl.DeviceIdType.MESH,
                ).wait_send()
        for rdma in amax_rdmas:
            rdma.wait_send()

    out_shape = jax.ShapeDtypeStruct((N_DEV * m, blk), jnp.float32)
    return pl.pallas_call(
        body,
        out_shape=out_shape,
        in_specs=[
            pl.BlockSpec(memory_space=pltpu.VMEM),
            pl.BlockSpec(memory_space=pltpu.VMEM),
        ],
        out_specs=pl.BlockSpec(memory_space=pltpu.VMEM),
        scratch_shapes=[
            pltpu.VMEM((N_DEV, m, blk), jnp.bfloat16),
            pltpu.VMEM((N_DEV, m, blk), jnp.bfloat16),
            pltpu.VMEM((1, 128), jnp.float32),
            pltpu.VMEM((N_DEV, 128), jnp.float32),
            pltpu.SemaphoreType.DMA,
            pltpu.SemaphoreType.DMA((N_DEV,)),
            pltpu.SemaphoreType.DMA((N_DEV,)),
            pltpu.SemaphoreType.DMA((N_DEV,)),
            pltpu.SemaphoreType.DMA((N_DEV,)),
        ],
        compiler_params=pltpu.CompilerParams(
            vmem_limit_bytes=60 * 1024 * 1024,
            collective_id=0,
        ),
    )(x, w_mat)
